# Initial kernel scaffold; baseline (speedup 1.0000x reference)
import jax
import jax.numpy as jnp
from jax.experimental import pallas as pl


def kernel(x, edge_index, W1, b1, W2, b2):
    raise NotImplementedError("write your pallas kernel here")


# speedup vs baseline: 10.8875x; 10.8875x over previous
import functools

import jax
import jax.numpy as jnp
from jax import lax
from jax.experimental import pallas as pl
from jax.experimental.pallas import tpu as pltpu
from jax.experimental.pallas import tpu_sc as plsc

N = 10000
E = 320000
NPAD = 10240
NC = 2
NS = 16
NTILES = NC * NS
EPT = E // NTILES
CH = 80
NCHUNK = EPT // CH
ROWS_PT = NPAD // NS

_MESH = plsc.VectorSubcoreMesh(core_axis_name="c", subcore_axis_name="s")
_SC_PARAMS = pltpu.CompilerParams(use_tc_tiling_on_sc=False)


def _deg_body(dst_hbm, ones_hbm, out_hbm, idx_d, ones_v, accum):
    c = lax.axis_index("c")
    s = lax.axis_index("s")
    rz = s * ROWS_PT
    pltpu.sync_copy(ones_hbm.at[pl.ds(CH, ROWS_PT)], accum.at[pl.ds(rz, ROWS_PT)])
    pltpu.sync_copy(ones_hbm.at[pl.ds(0, CH)], ones_v)
    plsc.subcore_barrier()
    tile = c * NS + s
    base = tile * EPT

    def chunk(k, carry):
        off = base + k * CH
        pltpu.sync_copy(dst_hbm.at[pl.ds(off, CH)], idx_d)
        pltpu.sync_copy(ones_v, accum.at[idx_d], add=True)
        return carry

    lax.fori_loop(0, NCHUNK, chunk, 0)
    plsc.subcore_barrier()
    pltpu.sync_copy(accum.at[pl.ds(rz, ROWS_PT)], out_hbm.at[c].at[pl.ds(rz, ROWS_PT)])


_deg_kernel = pl.kernel(
    _deg_body,
    out_type=jax.ShapeDtypeStruct((NC, NPAD, 16), jnp.float32),
    mesh=_MESH,
    scratch_types=[
        pltpu.VMEM((CH,), jnp.int32),
        pltpu.VMEM((CH, 16), jnp.float32),
        pltpu.VMEM_SHARED((NPAD, 16), jnp.float32),
    ],
    compiler_params=_SC_PARAMS,
)


def _prop_body(u_hbm, src_hbm, dst_hbm, zeros_hbm, out_hbm,
               idx_s, idx_d, buf, accum, sem):
    c = lax.axis_index("c")
    s = lax.axis_index("s")
    rz = s * ROWS_PT
    pltpu.sync_copy(zeros_hbm.at[pl.ds(rz, ROWS_PT)], accum.at[pl.ds(rz, ROWS_PT)])
    plsc.subcore_barrier()
    tile = c * NS + s
    base = tile * EPT

    def chunk(k, carry):
        off = base + k * CH
        pltpu.sync_copy(src_hbm.at[pl.ds(off, CH)], idx_s)
        pltpu.sync_copy(dst_hbm.at[pl.ds(off, CH)], idx_d)
        pltpu.async_copy(u_hbm.at[idx_s], buf, sem).wait()
        pltpu.sync_copy(buf, accum.at[idx_d], add=True)
        return carry

    lax.fori_loop(0, NCHUNK, chunk, 0)
    plsc.subcore_barrier()
    pltpu.sync_copy(accum.at[pl.ds(rz, ROWS_PT)], out_hbm.at[c].at[pl.ds(rz, ROWS_PT)])


def _make_prop(chan):
    return pl.kernel(
        _prop_body,
        out_type=jax.ShapeDtypeStruct((NC, NPAD, chan), jnp.float32),
        mesh=_MESH,
        scratch_types=[
            pltpu.VMEM((CH,), jnp.int32),
            pltpu.VMEM((CH,), jnp.int32),
            pltpu.VMEM((CH, chan), jnp.float32),
            pltpu.VMEM_SHARED((NPAD, chan), jnp.float32),
            pltpu.SemaphoreType.DMA,
        ],
        compiler_params=_SC_PARAMS,
    )


_prop128 = _make_prop(128)
_prop64 = _make_prop(64)


def _dinvs(degp_ref):
    deg = degp_ref[0] + degp_ref[1] + 1.0
    dinv = lax.rsqrt(deg)[:, 0:1]
    dinv2 = (1.0 / deg)[:, 0:1]
    return dinv, dinv2


def _k1_body(degp_ref, x_ref, w1_ref, u1_ref):
    dinv, _ = _dinvs(degp_ref)
    u1_ref[...] = dinv * jnp.dot(x_ref[...], w1_ref[...],
                                 preferred_element_type=jnp.float32)


def _k2_body(degp_ref, sp_ref, u_ref, w_ref):
    _, dinv2 = _dinvs(degp_ref)
    w_ref[...] = dinv2 * (sp_ref[0] + sp_ref[1] + u_ref[...])


def _k3_body(degp_ref, sp_ref, w1_ref, b1_ref, w2mat_ref, u2_ref):
    dinv, _ = _dinvs(degp_ref)
    y1 = sp_ref[0] + sp_ref[1] + w1_ref[...]
    hid = jnp.maximum(dinv * y1 + b1_ref[...], 0.0)
    u2_ref[...] = dinv * jnp.dot(hid, w2mat_ref[...],
                                 preferred_element_type=jnp.float32)


def _k5_body(degp_ref, sp_ref, w2_ref, b2_ref, o_ref):
    dinv, _ = _dinvs(degp_ref)
    o = dinv * (sp_ref[0] + sp_ref[1] + w2_ref[...]) + b2_ref[...]
    m = jnp.max(o, axis=1, keepdims=True)
    lse = jnp.log(jnp.sum(jnp.exp(o - m), axis=1, keepdims=True))
    o_ref[...] = o - m - lse


def _tc_call(body, out_chan):
    return pl.pallas_call(
        body, out_shape=jax.ShapeDtypeStruct((NPAD, out_chan), jnp.float32))


@jax.jit
def kernel(x, edge_index, W1, b1, W2, b2):
    src = edge_index[0]
    dst = edge_index[1]
    x_pad = jnp.pad(x, ((0, NPAD - N), (0, 0)))
    ones16 = jnp.ones((CH + ROWS_PT, 16), jnp.float32)
    ones16 = ones16.at[CH:].set(0.0)
    zeros128 = jnp.zeros((NPAD, 128), jnp.float32)
    zeros64 = jnp.zeros((NPAD, 64), jnp.float32)
    b1r = b1.reshape(1, 128)
    b2r = b2.reshape(1, 64)

    degp = _deg_kernel(dst, ones16)

    u1 = _tc_call(_k1_body, 128)(degp, x_pad, W1)
    s1 = _prop128(u1, src, dst, zeros128)
    w1 = _tc_call(_k2_body, 128)(degp, s1, u1)
    s2 = _prop128(w1, src, dst, zeros128)
    u2 = _tc_call(_k3_body, 64)(degp, s2, w1, b1r, W2)
    s3 = _prop64(u2, src, dst, zeros64)
    w2 = _tc_call(_k2_body, 64)(degp, s3, u2)
    s4 = _prop64(w2, src, dst, zeros64)
    o = _tc_call(_k5_body, 64)(degp, s4, w2, b2r)
    return o[:N]


# speedup vs baseline: 28.9867x; 2.6624x over previous
import functools

import jax
import jax.numpy as jnp
from jax import lax
from jax.experimental import pallas as pl
from jax.experimental.pallas import tpu as pltpu
from jax.experimental.pallas import tpu_sc as plsc

N = 10000
E = 320000
NPAD = 10240
NC = 2
NS = 16
NTILES = NC * NS
EPT = E // NTILES
CH = 80
NCHUNK = EPT // CH
ROWS_PT = NPAD // NS

_MESH = plsc.VectorSubcoreMesh(core_axis_name="c", subcore_axis_name="s")
_SC_PARAMS = pltpu.CompilerParams(use_tc_tiling_on_sc=False)
_SC_PARAMS_NOLAYOUT = pltpu.CompilerParams(use_tc_tiling_on_sc=False,
                                           needs_layout_passes=False)


def _deg_body(dst_hbm, ones_hbm, out_hbm, idx_d, ones_v, accum):
    c = lax.axis_index("c")
    s = lax.axis_index("s")
    rz = s * ROWS_PT
    pltpu.sync_copy(ones_hbm.at[pl.ds(CH, ROWS_PT)], accum.at[pl.ds(rz, ROWS_PT)])
    pltpu.sync_copy(ones_hbm.at[pl.ds(0, CH)], ones_v)
    tile = c * NS + s
    pltpu.sync_copy(dst_hbm.at[tile], idx_d)
    plsc.subcore_barrier()

    def chunk(k, carry):
        pltpu.sync_copy(ones_v, accum.at[idx_d.at[k]], add=True)
        return carry

    lax.fori_loop(0, NCHUNK, chunk, 0)
    plsc.subcore_barrier()
    pltpu.sync_copy(accum.at[pl.ds(rz, ROWS_PT)], out_hbm.at[c].at[pl.ds(rz, ROWS_PT)])


_deg_kernel = pl.kernel(
    _deg_body,
    out_type=jax.ShapeDtypeStruct((NC, NPAD, 16), jnp.float32),
    mesh=_MESH,
    scratch_types=[
        pltpu.VMEM((NCHUNK, CH), jnp.int32),
        pltpu.VMEM((CH, 16), jnp.float32),
        pltpu.VMEM_SHARED((NPAD, 16), jnp.float32),
    ],
    compiler_params=_SC_PARAMS,
)


def _prop_body(nbuf, u_hbm, src_hbm, dst_hbm, zeros_hbm, out_hbm,
               idx_s, idx_d, accum, *rest):
    bufs, sems = rest[:nbuf], rest[nbuf:]
    c = lax.axis_index("c")
    s = lax.axis_index("s")
    rz = s * ROWS_PT
    pltpu.sync_copy(zeros_hbm.at[pl.ds(rz, ROWS_PT)], accum.at[pl.ds(rz, ROWS_PT)])
    tile = c * NS + s
    pltpu.sync_copy(src_hbm.at[tile], idx_s)
    pltpu.sync_copy(dst_hbm.at[tile], idx_d)
    plsc.subcore_barrier()

    for b in range(nbuf):
        pltpu.async_copy(u_hbm.at[idx_s.at[b]], bufs[b], sems[b])

    def step(k, b):
        pltpu.make_async_copy(u_hbm.at[idx_s.at[k]], bufs[b], sems[b]).wait()
        pltpu.sync_copy(bufs[b], accum.at[idx_d.at[k]], add=True)
        nk = k + nbuf

        @pl.when(nk < NCHUNK)
        def _():
            pltpu.async_copy(u_hbm.at[idx_s.at[nk]], bufs[b], sems[b])

    def group(grp, carry):
        for b in range(nbuf):
            step(grp * nbuf + b, b)
        return carry

    ngrp = NCHUNK // nbuf
    lax.fori_loop(0, ngrp, group, 0)
    for r in range(ngrp * nbuf, NCHUNK):
        step(r, r % nbuf)
    plsc.subcore_barrier()
    pltpu.sync_copy(accum.at[pl.ds(rz, ROWS_PT)], out_hbm.at[c].at[pl.ds(rz, ROWS_PT)])


def _make_prop(chan, nbuf):
    return pl.kernel(
        functools.partial(_prop_body, nbuf),
        out_type=jax.ShapeDtypeStruct((NC, NPAD, chan), jnp.float32),
        mesh=_MESH,
        scratch_types=[
            pltpu.VMEM((NCHUNK, CH), jnp.int32),
            pltpu.VMEM((NCHUNK, CH), jnp.int32),
            pltpu.VMEM_SHARED((NPAD, chan), jnp.float32),
        ]
        + [pltpu.VMEM((CH, chan), jnp.float32) for _ in range(nbuf)]
        + [pltpu.SemaphoreType.DMA for _ in range(nbuf)],
        compiler_params=_SC_PARAMS,
    )


_prop128 = _make_prop(128, 2)
_prop64 = _make_prop(64, 5)


def _k1_body(degp_ref, x_ref, w1_ref, u1_ref, dinv_ref, dinv2_ref):
    deg = (degp_ref[0] + degp_ref[1])[:, 0:1] + 1.0
    deg = jnp.broadcast_to(deg, (NPAD, 8))
    dinv_ref[...] = lax.rsqrt(deg)
    dinv2_ref[...] = 1.0 / deg
    u1_ref[...] = dinv_ref[:, 0:1] * jnp.dot(x_ref[...], w1_ref[...],
                                             preferred_element_type=jnp.float32)


def _k2_body(dinv2_ref, sp_ref, u_ref, w_ref):
    w_ref[...] = dinv2_ref[:, 0:1] * (sp_ref[0] + sp_ref[1] + u_ref[...])


def _k3_body(dinv_ref, sp_ref, w1_ref, b1_ref, w2mat_ref, u2_ref):
    dinv = dinv_ref[:, 0:1]
    y1 = sp_ref[0] + sp_ref[1] + w1_ref[...]
    hid = jnp.maximum(dinv * y1 + b1_ref[...], 0.0)
    u2_ref[...] = dinv * jnp.dot(hid, w2mat_ref[...],
                                 preferred_element_type=jnp.float32)


def _k5_body(dinv_ref, sp_ref, w2_ref, b2_ref, o_ref):
    o = dinv_ref[:, 0:1] * (sp_ref[0] + sp_ref[1] + w2_ref[...]) + b2_ref[...]
    m = jnp.max(o, axis=1, keepdims=True)
    lse = jnp.log(jnp.sum(jnp.exp(o - m), axis=1, keepdims=True))
    o_ref[...] = o - m - lse


def _tc_call(body, out_chan):
    if isinstance(out_chan, tuple):
        out_shape = tuple(jax.ShapeDtypeStruct((NPAD, c), jnp.float32)
                          for c in out_chan)
    else:
        out_shape = jax.ShapeDtypeStruct((NPAD, out_chan), jnp.float32)
    return pl.pallas_call(body, out_shape=out_shape)


@jax.jit
def kernel(x, edge_index, W1, b1, W2, b2):
    src = edge_index[0].reshape(NTILES, NCHUNK, CH)
    dst = edge_index[1].reshape(NTILES, NCHUNK, CH)
    x_pad = jnp.pad(x, ((0, NPAD - N), (0, 0)))
    ones16 = jnp.ones((CH + ROWS_PT, 16), jnp.float32)
    ones16 = ones16.at[CH:].set(0.0)
    zeros128 = jnp.zeros((NPAD, 128), jnp.float32)
    zeros64 = jnp.zeros((NPAD, 64), jnp.float32)
    b1r = b1.reshape(1, 128)
    b2r = b2.reshape(1, 64)

    degp = _deg_kernel(dst, ones16)

    u1, dinv, dinv2 = _tc_call(_k1_body, (128, 8, 8))(degp, x_pad, W1)
    s1 = _prop128(u1, src, dst, zeros128)
    w1 = _tc_call(_k2_body, 128)(dinv2, s1, u1)
    s2 = _prop128(w1, src, dst, zeros128)
    u2 = _tc_call(_k3_body, 64)(dinv, s2, w1, b1r, W2)
    s3 = _prop64(u2, src, dst, zeros64)
    w2 = _tc_call(_k2_body, 64)(dinv2, s3, u2)
    s4 = _prop64(w2, src, dst, zeros64)
    o = _tc_call(_k5_body, 64)(dinv, s4, w2, b2r)
    return o[:N]


# speedup vs baseline: 31.8717x; 1.0995x over previous
import functools

import jax
import jax.numpy as jnp
from jax import lax
from jax.experimental import pallas as pl
from jax.experimental.pallas import tpu as pltpu
from jax.experimental.pallas import tpu_sc as plsc

N = 10000
E = 320000
NPAD = 10240
NC = 2
NS = 16
NTILES = NC * NS
EPT = E // NTILES
CH = 80
NCHUNK = EPT // CH
ROWS_PT = NPAD // NS

_MESH = plsc.VectorSubcoreMesh(core_axis_name="c", subcore_axis_name="s")
_SC_PARAMS = pltpu.CompilerParams(use_tc_tiling_on_sc=False)
_SC_PARAMS_NOLAYOUT = pltpu.CompilerParams(use_tc_tiling_on_sc=False,
                                           needs_layout_passes=False)


def _deg_body(dst_hbm, ones_hbm, out_hbm, idx_d, ones_v, accum):
    c = lax.axis_index("c")
    s = lax.axis_index("s")
    rz = s * ROWS_PT
    pltpu.sync_copy(ones_hbm.at[pl.ds(CH, ROWS_PT)], accum.at[pl.ds(rz, ROWS_PT)])
    pltpu.sync_copy(ones_hbm.at[pl.ds(0, CH)], ones_v)
    tile = c * NS + s
    pltpu.sync_copy(dst_hbm.at[tile], idx_d)
    plsc.subcore_barrier()

    def chunk(k, carry):
        pltpu.sync_copy(ones_v, accum.at[idx_d.at[k]], add=True)
        return carry

    lax.fori_loop(0, NCHUNK, chunk, 0)
    plsc.subcore_barrier()
    pltpu.sync_copy(accum.at[pl.ds(rz, ROWS_PT)], out_hbm.at[c].at[pl.ds(rz, ROWS_PT)])


_deg_kernel = pl.kernel(
    _deg_body,
    out_type=jax.ShapeDtypeStruct((NC, NPAD, 16), jnp.float32),
    mesh=_MESH,
    scratch_types=[
        pltpu.VMEM((NCHUNK, CH), jnp.int32),
        pltpu.VMEM((CH, 16), jnp.float32),
        pltpu.VMEM_SHARED((NPAD, 16), jnp.float32),
    ],
    compiler_params=_SC_PARAMS,
)


def _prop_body(nbuf, u_hbm, src_hbm, dst_hbm, zeros_hbm, out_hbm,
               idx_s, accum, *rest):
    bufs = rest[:nbuf]
    dbufs = rest[nbuf:2 * nbuf]
    sems = rest[2 * nbuf:3 * nbuf]
    dsems = rest[3 * nbuf:4 * nbuf]
    c = lax.axis_index("c")
    s = lax.axis_index("s")
    rz = s * ROWS_PT
    pltpu.sync_copy(zeros_hbm, accum.at[pl.ds(rz, ROWS_PT)])
    tile = c * NS + s
    pltpu.sync_copy(src_hbm.at[tile], idx_s)
    plsc.subcore_barrier()

    def issue(k, b):
        pltpu.async_copy(u_hbm.at[idx_s.at[k]], bufs[b], sems[b])
        pltpu.async_copy(dst_hbm.at[tile].at[k], dbufs[b], dsems[b])

    for b in range(nbuf):
        issue(b, b)

    def step(k, b):
        pltpu.make_async_copy(u_hbm.at[idx_s.at[k]], bufs[b], sems[b]).wait()
        pltpu.make_async_copy(dst_hbm.at[tile].at[k], dbufs[b], dsems[b]).wait()
        pltpu.sync_copy(bufs[b], accum.at[dbufs[b]], add=True)
        nk = k + nbuf

        @pl.when(nk < NCHUNK)
        def _():
            issue(nk, b)

    def group(grp, carry):
        for b in range(nbuf):
            step(grp * nbuf + b, b)
        return carry

    ngrp = NCHUNK // nbuf
    lax.fori_loop(0, ngrp, group, 0)
    for r in range(ngrp * nbuf, NCHUNK):
        step(r, r % nbuf)
    plsc.subcore_barrier()
    pltpu.sync_copy(accum.at[pl.ds(rz, ROWS_PT)], out_hbm.at[c].at[pl.ds(rz, ROWS_PT)])


def _make_prop(chan, nbuf):
    return pl.kernel(
        functools.partial(_prop_body, nbuf),
        out_type=jax.ShapeDtypeStruct((NC, NPAD, chan), jnp.float32),
        mesh=_MESH,
        scratch_types=[
            pltpu.VMEM((NCHUNK, CH), jnp.int32),
            pltpu.VMEM_SHARED((NPAD, chan), jnp.float32),
        ]
        + [pltpu.VMEM((CH, chan), jnp.float32) for _ in range(nbuf)]
        + [pltpu.VMEM((CH,), jnp.int32) for _ in range(nbuf)]
        + [pltpu.SemaphoreType.DMA for _ in range(2 * nbuf)],
        compiler_params=_SC_PARAMS,
    )


_prop128 = _make_prop(128, 3)
_prop64 = _make_prop(64, 5)


def _k1_body(degp_ref, x_ref, w1_ref, u1_ref, dinv_ref, dinv2_ref):
    deg = (degp_ref[0] + degp_ref[1])[:, 0:1] + 1.0
    deg = jnp.broadcast_to(deg, (NPAD, 8))
    dinv_ref[...] = lax.rsqrt(deg)
    dinv2_ref[...] = 1.0 / deg
    u1_ref[...] = dinv_ref[:, 0:1] * jnp.dot(x_ref[...], w1_ref[...],
                                             preferred_element_type=jnp.float32)


def _k2_body(dinv2_ref, sp_ref, u_ref, w_ref):
    w_ref[...] = dinv2_ref[:, 0:1] * (sp_ref[0] + sp_ref[1] + u_ref[...])


def _k3_body(dinv_ref, sp_ref, w1_ref, b1_ref, w2mat_ref, u2_ref):
    dinv = dinv_ref[:, 0:1]
    y1 = sp_ref[0] + sp_ref[1] + w1_ref[...]
    hid = jnp.maximum(dinv * y1 + b1_ref[...], 0.0)
    u2_ref[...] = dinv * jnp.dot(hid, w2mat_ref[...],
                                 preferred_element_type=jnp.float32)


def _k5_body(dinv_ref, sp_ref, w2_ref, b2_ref, o_ref):
    o = dinv_ref[:, 0:1] * (sp_ref[0] + sp_ref[1] + w2_ref[...]) + b2_ref[...]
    m = jnp.max(o, axis=1, keepdims=True)
    lse = jnp.log(jnp.sum(jnp.exp(o - m), axis=1, keepdims=True))
    o_ref[...] = o - m - lse


def _tc_call(body, out_chan):
    if isinstance(out_chan, tuple):
        out_shape = tuple(jax.ShapeDtypeStruct((NPAD, c), jnp.float32)
                          for c in out_chan)
    else:
        out_shape = jax.ShapeDtypeStruct((NPAD, out_chan), jnp.float32)
    return pl.pallas_call(body, out_shape=out_shape)


@jax.jit
def kernel(x, edge_index, W1, b1, W2, b2):
    src = edge_index[0].reshape(NTILES, NCHUNK, CH)
    dst = edge_index[1].reshape(NTILES, NCHUNK, CH)
    x_pad = jnp.pad(x, ((0, NPAD - N), (0, 0)))
    ones16 = jnp.ones((CH + ROWS_PT, 16), jnp.float32)
    ones16 = ones16.at[CH:].set(0.0)
    zeros128 = jnp.zeros((ROWS_PT, 128), jnp.float32)
    zeros64 = jnp.zeros((ROWS_PT, 64), jnp.float32)
    b1r = b1.reshape(1, 128)
    b2r = b2.reshape(1, 64)

    degp = _deg_kernel(dst, ones16)

    u1, dinv, dinv2 = _tc_call(_k1_body, (128, 8, 8))(degp, x_pad, W1)
    s1 = _prop128(u1, src, dst, zeros128)
    w1 = _tc_call(_k2_body, 128)(dinv2, s1, u1)
    s2 = _prop128(w1, src, dst, zeros128)
    u2 = _tc_call(_k3_body, 64)(dinv, s2, w1, b1r, W2)
    s3 = _prop64(u2, src, dst, zeros64)
    w2 = _tc_call(_k2_body, 64)(dinv2, s3, u2)
    s4 = _prop64(w2, src, dst, zeros64)
    o = _tc_call(_k5_body, 64)(dinv, s4, w2, b2r)
    return o[:N]


# speedup vs baseline: 34.4900x; 1.0822x over previous
import functools

import jax
import jax.numpy as jnp
from jax import lax
from jax.experimental import pallas as pl
from jax.experimental.pallas import tpu as pltpu
from jax.experimental.pallas import tpu_sc as plsc

N = 10000
E = 320000
NPAD = 10240
NC = 2
NS = 16
NTILES = NC * NS
EPT = E // NTILES
CH = 80
NCHUNK = EPT // CH
ROWS_PT = NPAD // NS

_MESH = plsc.VectorSubcoreMesh(core_axis_name="c", subcore_axis_name="s")
_SC_PARAMS = pltpu.CompilerParams(use_tc_tiling_on_sc=False)
_SC_PARAMS_NOLAYOUT = pltpu.CompilerParams(use_tc_tiling_on_sc=False,
                                           needs_layout_passes=False)


def _deg_body(edge_hbm, ones_hbm, out_hbm, idx_d, ones_v, accum):
    c = lax.axis_index("c")
    s = lax.axis_index("s")
    rz = s * ROWS_PT
    pltpu.sync_copy(ones_hbm.at[pl.ds(CH, ROWS_PT)], accum.at[pl.ds(rz, ROWS_PT)])
    pltpu.sync_copy(ones_hbm.at[pl.ds(0, CH)], ones_v)
    tile = c * NS + s
    pltpu.sync_copy(edge_hbm.at[1].at[tile], idx_d)
    plsc.subcore_barrier()

    def chunk(k, carry):
        pltpu.sync_copy(ones_v, accum.at[idx_d.at[k]], add=True)
        return carry

    lax.fori_loop(0, NCHUNK, chunk, 0)
    plsc.subcore_barrier()
    pltpu.sync_copy(accum.at[pl.ds(rz, ROWS_PT)],
                    out_hbm.at[c].at[pl.ds(rz, ROWS_PT), pl.ds(0, 16)])


_deg_kernel = pl.kernel(
    _deg_body,
    out_type=jax.ShapeDtypeStruct((NC, NPAD, 128), jnp.float32),
    mesh=_MESH,
    scratch_types=[
        pltpu.VMEM((NCHUNK, CH), jnp.int32),
        pltpu.VMEM((CH, 16), jnp.float32),
        pltpu.VMEM_SHARED((NPAD, 16), jnp.float32),
    ],
    compiler_params=_SC_PARAMS,
)


def _prop_body(nbuf, chan, u_hbm, edge_hbm, zeros_hbm, out_hbm,
               idx_s, accum, *rest):
    src_hbm = edge_hbm.at[0]
    dst_hbm = edge_hbm.at[1]
    bufs = rest[:nbuf]
    dbufs = rest[nbuf:2 * nbuf]
    sems = rest[2 * nbuf:3 * nbuf]
    dsems = rest[3 * nbuf:4 * nbuf]
    c = lax.axis_index("c")
    s = lax.axis_index("s")
    rz = s * ROWS_PT
    pltpu.sync_copy(zeros_hbm, accum.at[pl.ds(rz, ROWS_PT)])
    tile = c * NS + s
    pltpu.sync_copy(src_hbm.at[tile], idx_s)
    plsc.subcore_barrier()

    def issue(k, b):
        pltpu.async_copy(u_hbm.at[idx_s.at[k]], bufs[b], sems[b])
        pltpu.async_copy(dst_hbm.at[tile].at[k], dbufs[b], dsems[b])

    for b in range(nbuf):
        issue(b, b)

    def step(k, b):
        pltpu.make_async_copy(u_hbm.at[idx_s.at[k]], bufs[b], sems[b]).wait()
        pltpu.make_async_copy(dst_hbm.at[tile].at[k], dbufs[b], dsems[b]).wait()
        pltpu.sync_copy(bufs[b], accum.at[dbufs[b]], add=True)
        nk = k + nbuf

        @pl.when(nk < NCHUNK)
        def _():
            issue(nk, b)

    def group(grp, carry):
        for b in range(nbuf):
            step(grp * nbuf + b, b)
        return carry

    ngrp = NCHUNK // nbuf
    lax.fori_loop(0, ngrp, group, 0)
    for r in range(ngrp * nbuf, NCHUNK):
        step(r, r % nbuf)
    plsc.subcore_barrier()
    pltpu.sync_copy(accum.at[pl.ds(rz, ROWS_PT)],
                    out_hbm.at[c].at[pl.ds(rz, ROWS_PT), pl.ds(0, chan)])


def _make_prop(chan, nbuf):
    return pl.kernel(
        functools.partial(_prop_body, nbuf, chan),
        out_type=jax.ShapeDtypeStruct((NC, NPAD, 128), jnp.float32),
        mesh=_MESH,
        scratch_types=[
            pltpu.VMEM((NCHUNK, CH), jnp.int32),
            pltpu.VMEM_SHARED((NPAD, chan), jnp.float32),
        ]
        + [pltpu.VMEM((CH, chan), jnp.float32) for _ in range(nbuf)]
        + [pltpu.VMEM((CH,), jnp.int32) for _ in range(nbuf)]
        + [pltpu.SemaphoreType.DMA for _ in range(2 * nbuf)],
        compiler_params=_SC_PARAMS,
    )


_prop128 = _make_prop(128, 3)
_prop64 = _make_prop(64, 5)


def _k0_body(x_ref, w1_ref, t_ref):
    t_ref[...] = jnp.dot(x_ref[...], w1_ref[...],
                         preferred_element_type=jnp.float32)


def _k1_body(degp_ref, t_ref, u1_ref, dinv_ref, dinv2_ref):
    deg = (degp_ref[0, :, 0:1] + degp_ref[1, :, 0:1]) + 1.0
    deg = jnp.broadcast_to(deg, (NPAD, 8))
    dinv_ref[...] = lax.rsqrt(deg)
    dinv2_ref[...] = 1.0 / deg
    u1_ref[0:N] = dinv_ref[0:N, 0:1] * t_ref[...]
    u1_ref[N:NPAD] = jnp.zeros((NPAD - N, 128), jnp.float32)


def _k2_body(dinv2_ref, sp_ref, u_ref, w_ref):
    chan = u_ref.shape[1]
    sp = sp_ref[0, :, 0:chan] + sp_ref[1, :, 0:chan]
    w_ref[...] = dinv2_ref[:, 0:1] * (sp + u_ref[...])


def _k3_body(dinv_ref, sp_ref, w1_ref, b1_ref, w2mat_ref, u2_ref):
    dinv = dinv_ref[:, 0:1]
    y1 = sp_ref[0, :, 0:128] + sp_ref[1, :, 0:128] + w1_ref[...]
    hid = jnp.maximum(dinv * y1 + b1_ref[...], 0.0)
    u2_ref[...] = dinv * jnp.dot(hid, w2mat_ref[...],
                                 preferred_element_type=jnp.float32)


def _k5_body(dinv_ref, sp_ref, w2_ref, b2_ref, o_ref):
    sp = sp_ref[0, 0:N, 0:64] + sp_ref[1, 0:N, 0:64]
    o = dinv_ref[0:N, 0:1] * (sp + w2_ref[0:N]) + b2_ref[...]
    m = jnp.max(o, axis=1, keepdims=True)
    lse = jnp.log(jnp.sum(jnp.exp(o - m), axis=1, keepdims=True))
    o_ref[...] = o - m - lse


def _tc_call(body, out_chan):
    if isinstance(out_chan, tuple):
        out_shape = tuple(jax.ShapeDtypeStruct((NPAD, c), jnp.float32)
                          for c in out_chan)
    else:
        out_shape = jax.ShapeDtypeStruct((NPAD, out_chan), jnp.float32)
    return pl.pallas_call(body, out_shape=out_shape)


@jax.jit
def kernel(x, edge_index, W1, b1, W2, b2):
    edge3 = edge_index.reshape(2, NTILES, NCHUNK, CH)
    ones16 = jnp.ones((CH + ROWS_PT, 16), jnp.float32)
    ones16 = ones16.at[CH:].set(0.0)
    zeros128 = jnp.zeros((ROWS_PT, 128), jnp.float32)
    zeros64 = jnp.zeros((ROWS_PT, 64), jnp.float32)
    b1r = b1.reshape(1, 128)
    b2r = b2.reshape(1, 64)

    degp = _deg_kernel(edge3, ones16)
    t = pl.pallas_call(
        _k0_body, out_shape=jax.ShapeDtypeStruct((N, 128), jnp.float32))(x, W1)

    u1, dinv, dinv2 = _tc_call(_k1_body, (128, 8, 8))(degp, t)
    s1 = _prop128(u1, edge3, zeros128)
    w1 = _tc_call(_k2_body, 128)(dinv2, s1, u1)
    s2 = _prop128(w1, edge3, zeros128)
    u2 = _tc_call(_k3_body, 64)(dinv, s2, w1, b1r, W2)
    s3 = _prop64(u2, edge3, zeros64)
    w2 = _tc_call(_k2_body, 64)(dinv2, s3, u2)
    s4 = _prop64(w2, edge3, zeros64)
    o = pl.pallas_call(
        _k5_body, out_shape=jax.ShapeDtypeStruct((N, 64), jnp.float32))(
            dinv, s4, w2, b2r)
    return o


# speedup vs baseline: 35.1395x; 1.0188x over previous
import functools

import jax
import jax.numpy as jnp
from jax import lax
from jax.experimental import pallas as pl
from jax.experimental.pallas import tpu as pltpu
from jax.experimental.pallas import tpu_sc as plsc

N = 10000
E = 320000
NPAD = 10240
NC = 2
NS = 16
NTILES = NC * NS
EPT = E // NTILES
CH = 80
NCHUNK = EPT // CH
ROWS_PT = NPAD // NS

_MESH = plsc.VectorSubcoreMesh(core_axis_name="c", subcore_axis_name="s")
_SC_PARAMS = pltpu.CompilerParams(use_tc_tiling_on_sc=False)
_SC_PARAMS_NOLAYOUT = pltpu.CompilerParams(use_tc_tiling_on_sc=False,
                                           needs_layout_passes=False)


def _deg_body(edge_hbm, ones_hbm, out_hbm, idx_d, ones_v, accum, ssem):
    c = lax.axis_index("c")
    s = lax.axis_index("s")
    rz = s * ROWS_PT
    pltpu.sync_copy(ones_hbm.at[pl.ds(CH, ROWS_PT)], accum.at[pl.ds(rz, ROWS_PT)])
    pltpu.sync_copy(ones_hbm.at[pl.ds(0, CH)], ones_v)
    tile = c * NS + s
    pltpu.sync_copy(edge_hbm.at[1].at[tile], idx_d)
    plsc.subcore_barrier()

    def chunk(k, carry):
        pltpu.async_copy(ones_v, accum.at[idx_d.at[k]], ssem, add=True)
        return carry

    lax.fori_loop(0, NCHUNK, chunk, 0)

    def drain(k, carry):
        pltpu.make_async_copy(ones_v, accum.at[idx_d.at[0]], ssem).wait()
        return carry

    lax.fori_loop(0, NCHUNK, drain, 0)
    plsc.subcore_barrier()
    pltpu.sync_copy(accum.at[pl.ds(rz, ROWS_PT)],
                    out_hbm.at[c].at[pl.ds(rz, ROWS_PT), pl.ds(0, 16)])


_deg_kernel = pl.kernel(
    _deg_body,
    out_type=jax.ShapeDtypeStruct((NC, NPAD, 128), jnp.float32),
    mesh=_MESH,
    scratch_types=[
        pltpu.VMEM((NCHUNK, CH), jnp.int32),
        pltpu.VMEM((CH, 16), jnp.float32),
        pltpu.VMEM_SHARED((NPAD, 16), jnp.float32),
        pltpu.SemaphoreType.DMA,
    ],
    compiler_params=_SC_PARAMS,
)


def _prop_body(nbuf, chan, u_hbm, edge_hbm, zeros_hbm, out_hbm,
               idx_s, accum, *rest):
    src_hbm = edge_hbm.at[0]
    dst_hbm = edge_hbm.at[1]
    bufs = rest[:nbuf]
    dbufs = rest[nbuf:2 * nbuf]
    sems = rest[2 * nbuf:3 * nbuf]
    dsems = rest[3 * nbuf:4 * nbuf]
    c = lax.axis_index("c")
    s = lax.axis_index("s")
    rz = s * ROWS_PT
    pltpu.sync_copy(zeros_hbm, accum.at[pl.ds(rz, ROWS_PT)])
    tile = c * NS + s
    pltpu.sync_copy(src_hbm.at[tile], idx_s)
    plsc.subcore_barrier()

    def issue(k, b):
        pltpu.async_copy(u_hbm.at[idx_s.at[k]], bufs[b], sems[b])
        pltpu.async_copy(dst_hbm.at[tile].at[k], dbufs[b], dsems[b])

    for b in range(nbuf):
        issue(b, b)

    def step(k, b):
        pltpu.make_async_copy(u_hbm.at[idx_s.at[k]], bufs[b], sems[b]).wait()
        pltpu.make_async_copy(dst_hbm.at[tile].at[k], dbufs[b], dsems[b]).wait()
        pltpu.sync_copy(bufs[b], accum.at[dbufs[b]], add=True)
        nk = k + nbuf

        @pl.when(nk < NCHUNK)
        def _():
            issue(nk, b)

    def group(grp, carry):
        for b in range(nbuf):
            step(grp * nbuf + b, b)
        return carry

    ngrp = NCHUNK // nbuf
    lax.fori_loop(0, ngrp, group, 0)
    for r in range(ngrp * nbuf, NCHUNK):
        step(r, r % nbuf)
    plsc.subcore_barrier()
    pltpu.sync_copy(accum.at[pl.ds(rz, ROWS_PT)],
                    out_hbm.at[c].at[pl.ds(rz, ROWS_PT), pl.ds(0, chan)])


def _make_prop(chan, nbuf):
    return pl.kernel(
        functools.partial(_prop_body, nbuf, chan),
        out_type=jax.ShapeDtypeStruct((NC, NPAD, 128), jnp.float32),
        mesh=_MESH,
        scratch_types=[
            pltpu.VMEM((NCHUNK, CH), jnp.int32),
            pltpu.VMEM_SHARED((NPAD, chan), jnp.float32),
        ]
        + [pltpu.VMEM((CH, chan), jnp.float32) for _ in range(nbuf)]
        + [pltpu.VMEM((CH,), jnp.int32) for _ in range(nbuf)]
        + [pltpu.SemaphoreType.DMA for _ in range(2 * nbuf)],
        compiler_params=_SC_PARAMS,
    )


_prop128 = _make_prop(128, 3)
_prop64 = _make_prop(64, 5)


def _k0_body(x_ref, w1_ref, t_ref):
    t_ref[...] = jnp.dot(x_ref[...], w1_ref[...],
                         preferred_element_type=jnp.float32)


def _k1_body(degp_ref, t_ref, u1_ref, dinv_ref, dinv2_ref):
    deg = (degp_ref[0, :, 0:1] + degp_ref[1, :, 0:1]) + 1.0
    deg = jnp.broadcast_to(deg, (NPAD, 8))
    dinv_ref[...] = lax.rsqrt(deg)
    dinv2_ref[...] = 1.0 / deg
    u1_ref[0:N] = dinv_ref[0:N, 0:1] * t_ref[...]
    u1_ref[N:NPAD] = jnp.zeros((NPAD - N, 128), jnp.float32)


def _k2_body(dinv2_ref, sp_ref, u_ref, w_ref):
    chan = u_ref.shape[1]
    sp = sp_ref[0, :, 0:chan] + sp_ref[1, :, 0:chan]
    w_ref[...] = dinv2_ref[:, 0:1] * (sp + u_ref[...])


def _k3_body(dinv_ref, sp_ref, w1_ref, b1_ref, w2mat_ref, u2_ref):
    dinv = dinv_ref[:, 0:1]
    y1 = sp_ref[0, :, 0:128] + sp_ref[1, :, 0:128] + w1_ref[...]
    hid = jnp.maximum(dinv * y1 + b1_ref[...], 0.0)
    u2_ref[...] = dinv * jnp.dot(hid, w2mat_ref[...],
                                 preferred_element_type=jnp.float32)


def _k5_body(dinv_ref, sp_ref, w2_ref, b2_ref, o_ref):
    sp = sp_ref[0, 0:N, 0:64] + sp_ref[1, 0:N, 0:64]
    o = dinv_ref[0:N, 0:1] * (sp + w2_ref[0:N]) + b2_ref[...]
    m = jnp.max(o, axis=1, keepdims=True)
    lse = jnp.log(jnp.sum(jnp.exp(o - m), axis=1, keepdims=True))
    o_ref[...] = o - m - lse


def _tc_call(body, out_chan):
    if isinstance(out_chan, tuple):
        out_shape = tuple(jax.ShapeDtypeStruct((NPAD, c), jnp.float32)
                          for c in out_chan)
    else:
        out_shape = jax.ShapeDtypeStruct((NPAD, out_chan), jnp.float32)
    return pl.pallas_call(body, out_shape=out_shape)


@jax.jit
def kernel(x, edge_index, W1, b1, W2, b2):
    edge3 = edge_index.reshape(2, NTILES, NCHUNK, CH)
    ones16 = jnp.ones((CH + ROWS_PT, 16), jnp.float32)
    ones16 = ones16.at[CH:].set(0.0)
    zeros128 = jnp.zeros((ROWS_PT, 128), jnp.float32)
    zeros64 = jnp.zeros((ROWS_PT, 64), jnp.float32)
    b1r = b1.reshape(1, 128)
    b2r = b2.reshape(1, 64)

    degp = _deg_kernel(edge3, ones16)
    t = pl.pallas_call(
        _k0_body, out_shape=jax.ShapeDtypeStruct((N, 128), jnp.float32))(x, W1)

    u1, dinv, dinv2 = _tc_call(_k1_body, (128, 8, 8))(degp, t)
    s1 = _prop128(u1, edge3, zeros128)
    w1 = _tc_call(_k2_body, 128)(dinv2, s1, u1)
    s2 = _prop128(w1, edge3, zeros128)
    u2 = _tc_call(_k3_body, 64)(dinv, s2, w1, b1r, W2)
    s3 = _prop64(u2, edge3, zeros64)
    w2 = _tc_call(_k2_body, 64)(dinv2, s3, u2)
    s4 = _prop64(w2, edge3, zeros64)
    o = pl.pallas_call(
        _k5_body, out_shape=jax.ShapeDtypeStruct((N, 64), jnp.float32))(
            dinv, s4, w2, b2r)
    return o
